# Initial kernel scaffold; baseline (speedup 1.0000x reference)
import jax
import jax.numpy as jnp
from jax.experimental import pallas as pl


def kernel(x, edge_index, edge_weight, W1, b1, W2, b2, W3, b3, W4, b4):
    raise NotImplementedError("write your pallas kernel here")


# speedup vs baseline: 5.4733x; 5.4733x over previous
import jax
import jax.numpy as jnp
from jax import lax
from jax.experimental import pallas as pl
from jax.experimental.pallas import tpu as pltpu
from jax.experimental.pallas import tpu_sc as plsc

N = 10000
D = 128
E = 320000
NC = 2
NS = 16
NW = NC * NS
EPW = E // NW
CHUNK = 80
NCH = EPW // CHUNK
GRP = 25
NGRP = NCH // GRP
NPAD = 10240
ROWS_PER_SUB = NPAD // NS
ZROWS = 128
LANES = 16


def _sc_body(support_hbm, dst_hbm, src_hbm, ew_hbm, out_hbm,
             src_v, dst_v, ew_v, rows_v, acc_sh, sem):
    c = lax.axis_index("c")
    s = lax.axis_index("s")
    wid = c * NS + s

    def zrow(r, carry):
        for k in range(D // LANES):
            rows_v[r, pl.ds(k * LANES, LANES)] = jnp.zeros((LANES,), jnp.float32)
        return carry
    lax.fori_loop(0, CHUNK, zrow, 0)
    for t in range(ROWS_PER_SUB // CHUNK):
        pltpu.sync_copy(rows_v, acc_sh.at[pl.ds(s * ROWS_PER_SUB + t * CHUNK, CHUNK)])
    plsc.subcore_barrier()

    pltpu.sync_copy(ew_hbm.at[wid], ew_v)

    def group_body(gi, carry0):
        pltpu.sync_copy(src_hbm.at[wid, gi], src_v)
        pltpu.sync_copy(dst_hbm.at[wid, gi], dst_v)

        def chunk_body(j, carry):
            pltpu.async_copy(support_hbm.at[src_v.at[j]], rows_v, sem).wait()

            def grp_body(g, carry2):
                wv = ew_v[pl.ds((gi * GRP + j) * CHUNK + g * LANES, LANES)]
                for t in range(LANES):
                    w = wv[t]
                    e = g * LANES + t
                    for k in range(D // LANES):
                        sl = pl.ds(k * LANES, LANES)
                        rows_v[e, sl] = rows_v[e, sl] * w
                return carry2
            lax.fori_loop(0, CHUNK // LANES, grp_body, 0)

            pltpu.sync_copy(rows_v, acc_sh.at[dst_v.at[j]], add=True)
            return carry
        lax.fori_loop(0, GRP, chunk_body, 0)
        return carry0
    lax.fori_loop(0, NGRP, group_body, 0)

    plsc.subcore_barrier()
    pltpu.sync_copy(acc_sh.at[pl.ds(s * ROWS_PER_SUB, ROWS_PER_SUB)],
                    out_hbm.at[c, pl.ds(s * ROWS_PER_SUB, ROWS_PER_SUB)])


_sc_segment = pl.kernel(
    _sc_body,
    out_type=jax.ShapeDtypeStruct((NC, NPAD, D), jnp.float32),
    mesh=plsc.VectorSubcoreMesh(core_axis_name="c", subcore_axis_name="s"),
    scratch_types=[
        pltpu.VMEM((GRP, CHUNK), jnp.int32),
        pltpu.VMEM((GRP, CHUNK), jnp.int32),
        pltpu.VMEM((EPW,), jnp.float32),
        pltpu.VMEM((CHUNK, D), jnp.float32),
        pltpu.VMEM_SHARED((NPAD, D), jnp.float32),
        pltpu.SemaphoreType.DMA,
    ],
)


BM = 1000


def _mm_first_body(x_ref, w_ref, o_ref):
    o_ref[...] = jnp.dot(x_ref[...], w_ref[...],
                         preferred_element_type=jnp.float32,
                         precision=jax.lax.Precision.HIGHEST)


def _mm_fused_body(p0_ref, p1_ref, b_ref, w_ref, o_ref):
    h = p0_ref[...] + p1_ref[...] + b_ref[...]
    h = jnp.where(h >= 0, h, 0.25 * h)
    o_ref[...] = jnp.dot(h, w_ref[...],
                         preferred_element_type=jnp.float32,
                         precision=jax.lax.Precision.HIGHEST)


def _act_body(p0_ref, p1_ref, b_ref, o_ref):
    h = p0_ref[...] + p1_ref[...] + b_ref[...]
    o_ref[...] = jnp.where(h >= 0, h, 0.25 * h)


def _mm_first(x, W):
    return pl.pallas_call(
        _mm_first_body,
        grid=(N // BM,),
        in_specs=[pl.BlockSpec((BM, D), lambda i: (i, 0)),
                  pl.BlockSpec((D, D), lambda i: (0, 0))],
        out_specs=pl.BlockSpec((BM, D), lambda i: (i, 0)),
        out_shape=jax.ShapeDtypeStruct((N, D), jnp.float32),
    )(x, W)


def _mm_fused(p0, p1, b, W):
    return pl.pallas_call(
        _mm_fused_body,
        grid=(N // BM,),
        in_specs=[pl.BlockSpec((BM, D), lambda i: (i, 0)),
                  pl.BlockSpec((BM, D), lambda i: (i, 0)),
                  pl.BlockSpec((1, D), lambda i: (0, 0)),
                  pl.BlockSpec((D, D), lambda i: (0, 0))],
        out_specs=pl.BlockSpec((BM, D), lambda i: (i, 0)),
        out_shape=jax.ShapeDtypeStruct((N, D), jnp.float32),
    )(p0, p1, b.reshape(1, D), W)


def _act(p0, p1, b):
    return pl.pallas_call(
        _act_body,
        grid=(N // BM,),
        in_specs=[pl.BlockSpec((BM, D), lambda i: (i, 0)),
                  pl.BlockSpec((BM, D), lambda i: (i, 0)),
                  pl.BlockSpec((1, D), lambda i: (0, 0))],
        out_specs=pl.BlockSpec((BM, D), lambda i: (i, 0)),
        out_shape=jax.ShapeDtypeStruct((N, D), jnp.float32),
    )(p0, p1, b.reshape(1, D))


def kernel(x, edge_index, edge_weight, W1, b1, W2, b2, W3, b3, W4, b4):
    dst = edge_index[0].astype(jnp.int32).reshape(NW, NGRP, GRP, CHUNK)
    src = edge_index[1].astype(jnp.int32).reshape(NW, NGRP, GRP, CHUNK)
    ew = edge_weight.reshape(NW, EPW)

    def seg(support):
        p = _sc_segment(support, dst, src, ew)
        return p[0, :N], p[1, :N]

    s = _mm_first(x, W1)
    p0, p1 = seg(s)
    s = _mm_fused(p0, p1, b1, W2)
    p0, p1 = seg(s)
    s = _mm_fused(p0, p1, b2, W3)
    p0, p1 = seg(s)
    s = _mm_fused(p0, p1, b3, W4)
    p0, p1 = seg(s)
    return _act(p0, p1, b4)


# speedup vs baseline: 6.9380x; 1.2676x over previous
import jax
import jax.numpy as jnp
from jax import lax
from jax.experimental import pallas as pl
from jax.experimental.pallas import tpu as pltpu
from jax.experimental.pallas import tpu_sc as plsc

N = 10000
D = 128
E = 320000
NC = 2
NS = 16
NW = NC * NS
EPW = E // NW
CHUNK = 40
NCH = EPW // CHUNK
GRP = 50
NGRP = NCH // GRP
EWPAD = EPW + 16
NPAD = 10240
ROWS_PER_SUB = NPAD // NS
ZROWS = 128
LANES = 16


def _sc_body(support_hbm, dst_hbm, src_hbm, ew_hbm, out_hbm,
             src_v, dst_v, ew_v, rows0, rows1, acc_sh,
             sem_g0, sem_g1, sem_s0, sem_s1):
    c = lax.axis_index("c")
    s = lax.axis_index("s")
    wid = c * NS + s

    def zrow(r, carry):
        for k in range(D // LANES):
            rows0[r, pl.ds(k * LANES, LANES)] = jnp.zeros((LANES,), jnp.float32)
        return carry
    lax.fori_loop(0, CHUNK, zrow, 0)
    for t in range(ROWS_PER_SUB // CHUNK):
        pltpu.sync_copy(rows0, acc_sh.at[pl.ds(s * ROWS_PER_SUB + t * CHUNK, CHUNK)])
    plsc.subcore_barrier()

    pltpu.sync_copy(ew_hbm.at[wid], ew_v)

    def scale(rows, j):
        for g, off, lo in ((0, 0, 0), (1, 16, 0), (2, 24, 8)):
            wv = ew_v[pl.ds(j * CHUNK + off, LANES)]
            for t in range(lo, LANES):
                w = wv[t]
                e = off + t
                for k in range(D // LANES):
                    sl = pl.ds(k * LANES, LANES)
                    rows[e, sl] = rows[e, sl] * w

    def group_body(gi, carry0):
        pltpu.sync_copy(src_hbm.at[wid, gi], src_v)
        pltpu.sync_copy(dst_hbm.at[wid, gi], dst_v)
        pltpu.async_copy(support_hbm.at[src_v.at[0]], rows0, sem_g0)

        def pair_body(j2, carry):
            a = 2 * j2
            b = a + 1
            @pl.when(j2 > 0)
            def _():
                pltpu.make_async_copy(
                    rows1, acc_sh.at[dst_v.at[a - 1]], sem_s1).wait()
            pltpu.async_copy(support_hbm.at[src_v.at[b]], rows1, sem_g1)
            pltpu.make_async_copy(
                support_hbm.at[src_v.at[a]], rows0, sem_g0).wait()
            scale(rows0, gi * GRP + a)
            ca = pltpu.async_copy(rows0, acc_sh.at[dst_v.at[a]], sem_s0,
                                  add=True)
            pltpu.make_async_copy(
                support_hbm.at[src_v.at[b]], rows1, sem_g1).wait()
            scale(rows1, gi * GRP + b)
            ca.wait()

            @pl.when(j2 + 1 < GRP // 2)
            def _():
                pltpu.async_copy(support_hbm.at[src_v.at[a + 2]], rows0,
                                 sem_g0)
            pltpu.async_copy(rows1, acc_sh.at[dst_v.at[b]], sem_s1, add=True)
            return carry
        lax.fori_loop(0, GRP // 2, pair_body, 0)
        pltpu.make_async_copy(
            rows1, acc_sh.at[dst_v.at[GRP - 1]], sem_s1).wait()
        return carry0
    lax.fori_loop(0, NGRP, group_body, 0)

    plsc.subcore_barrier()
    pltpu.sync_copy(acc_sh.at[pl.ds(s * ROWS_PER_SUB, ROWS_PER_SUB)],
                    out_hbm.at[c, pl.ds(s * ROWS_PER_SUB, ROWS_PER_SUB)])


_sc_segment = pl.kernel(
    _sc_body,
    out_type=jax.ShapeDtypeStruct((NC, NPAD, D), jnp.float32),
    mesh=plsc.VectorSubcoreMesh(core_axis_name="c", subcore_axis_name="s"),
    scratch_types=[
        pltpu.VMEM((GRP, CHUNK), jnp.int32),
        pltpu.VMEM((GRP, CHUNK), jnp.int32),
        pltpu.VMEM((EPW,), jnp.float32),
        pltpu.VMEM((CHUNK, D), jnp.float32),
        pltpu.VMEM((CHUNK, D), jnp.float32),
        pltpu.VMEM_SHARED((NPAD, D), jnp.float32),
        pltpu.SemaphoreType.DMA,
        pltpu.SemaphoreType.DMA,
        pltpu.SemaphoreType.DMA,
        pltpu.SemaphoreType.DMA,
    ],
)


BM = 1000


def _mm_first_body(x_ref, w_ref, o_ref):
    o_ref[...] = jnp.dot(x_ref[...], w_ref[...],
                         preferred_element_type=jnp.float32,
                         precision=jax.lax.Precision.HIGHEST)


def _mm_fused_body(p0_ref, p1_ref, b_ref, w_ref, o_ref):
    h = p0_ref[...] + p1_ref[...] + b_ref[...]
    h = jnp.where(h >= 0, h, 0.25 * h)
    o_ref[...] = jnp.dot(h, w_ref[...],
                         preferred_element_type=jnp.float32,
                         precision=jax.lax.Precision.HIGHEST)


def _act_body(p0_ref, p1_ref, b_ref, o_ref):
    h = p0_ref[...] + p1_ref[...] + b_ref[...]
    o_ref[...] = jnp.where(h >= 0, h, 0.25 * h)


def _mm_first(x, W):
    return pl.pallas_call(
        _mm_first_body,
        grid=(N // BM,),
        in_specs=[pl.BlockSpec((BM, D), lambda i: (i, 0)),
                  pl.BlockSpec((D, D), lambda i: (0, 0))],
        out_specs=pl.BlockSpec((BM, D), lambda i: (i, 0)),
        out_shape=jax.ShapeDtypeStruct((N, D), jnp.float32),
    )(x, W)


def _mm_fused(p0, p1, b, W):
    return pl.pallas_call(
        _mm_fused_body,
        grid=(N // BM,),
        in_specs=[pl.BlockSpec((BM, D), lambda i: (i, 0)),
                  pl.BlockSpec((BM, D), lambda i: (i, 0)),
                  pl.BlockSpec((1, D), lambda i: (0, 0)),
                  pl.BlockSpec((D, D), lambda i: (0, 0))],
        out_specs=pl.BlockSpec((BM, D), lambda i: (i, 0)),
        out_shape=jax.ShapeDtypeStruct((N, D), jnp.float32),
    )(p0, p1, b.reshape(1, D), W)


def _act(p0, p1, b):
    return pl.pallas_call(
        _act_body,
        grid=(N // BM,),
        in_specs=[pl.BlockSpec((BM, D), lambda i: (i, 0)),
                  pl.BlockSpec((BM, D), lambda i: (i, 0)),
                  pl.BlockSpec((1, D), lambda i: (0, 0))],
        out_specs=pl.BlockSpec((BM, D), lambda i: (i, 0)),
        out_shape=jax.ShapeDtypeStruct((N, D), jnp.float32),
    )(p0, p1, b.reshape(1, D))


def kernel(x, edge_index, edge_weight, W1, b1, W2, b2, W3, b3, W4, b4):
    dst = edge_index[0].astype(jnp.int32).reshape(NW, NGRP, GRP, CHUNK)
    src = edge_index[1].astype(jnp.int32).reshape(NW, NGRP, GRP, CHUNK)
    ew = edge_weight.reshape(NW, EPW)

    def seg(support):
        p = _sc_segment(support, dst, src, ew)
        return p[0, :N], p[1, :N]

    s = _mm_first(x, W1)
    p0, p1 = seg(s)
    s = _mm_fused(p0, p1, b1, W2)
    p0, p1 = seg(s)
    s = _mm_fused(p0, p1, b2, W3)
    p0, p1 = seg(s)
    s = _mm_fused(p0, p1, b3, W4)
    p0, p1 = seg(s)
    return _act(p0, p1, b4)


# speedup vs baseline: 8.1402x; 1.1733x over previous
import jax
import jax.numpy as jnp
from jax import lax
from jax.experimental import pallas as pl
from jax.experimental.pallas import tpu as pltpu
from jax.experimental.pallas import tpu_sc as plsc

N = 10000
D = 128
E = 320000
NC = 2
NS = 16
NW = NC * NS
EPW = E // NW
CHUNK = 40
NCH = EPW // CHUNK
GRP = 50
NGRP = NCH // GRP
NBUF = 4
NPAD = 10240
ROWS_PER_SUB = NPAD // NS
LANES = 16


def _sc_body(support_hbm, dst_hbm, src_hbm, ew_hbm, out_hbm,
             src_v, dst_v, ew_v, rows, acc_sh, sem_g, sem_s):
    c = lax.axis_index("c")
    s = lax.axis_index("s")
    wid = c * NS + s

    with jax.named_scope("sc_zero"):
        def zrow(r, carry):
            for k in range(D // LANES):
                rows[0, r, pl.ds(k * LANES, LANES)] = jnp.zeros((LANES,),
                                                                jnp.float32)
            return carry
        lax.fori_loop(0, CHUNK, zrow, 0)
        for t in range(ROWS_PER_SUB // CHUNK):
            pltpu.sync_copy(
                rows.at[0],
                acc_sh.at[pl.ds(s * ROWS_PER_SUB + t * CHUNK, CHUNK)])
        plsc.subcore_barrier()

    with jax.named_scope("sc_stage_ew"):
        pltpu.sync_copy(ew_hbm.at[wid], ew_v)

    def scale(rows, j):
        for g, off, lo in ((0, 0, 0), (1, 16, 0), (2, 24, 8)):
            wv = ew_v[pl.ds(j * CHUNK + off, LANES)]
            for t in range(lo, LANES):
                w = wv[t]
                e = off + t
                for k in range(D // LANES):
                    sl = pl.ds(k * LANES, LANES)
                    rows[e, sl] = rows[e, sl] * w

    with jax.named_scope("sc_edges"):
        def group_body(gi, carry0):
            pltpu.sync_copy(src_hbm.at[wid, gi], src_v)
            pltpu.sync_copy(dst_hbm.at[wid, gi], dst_v)
            pltpu.async_copy(support_hbm.at[src_v.at[0]], rows.at[0],
                             sem_g.at[0])
            pltpu.async_copy(support_hbm.at[src_v.at[1]], rows.at[1],
                             sem_g.at[1])

            def chunk_body(j, carry):
                q = j % NBUF
                qn = (j + 2) % NBUF
                pltpu.make_async_copy(
                    support_hbm.at[src_v.at[j]], rows.at[q],
                    sem_g.at[q]).wait()
                scale(rows.at[q], gi * GRP + j)
                pltpu.async_copy(rows.at[q], acc_sh.at[dst_v.at[j]],
                                 sem_s.at[q], add=True)

                @pl.when(j >= 2)
                def _():
                    pltpu.make_async_copy(
                        rows.at[qn], acc_sh.at[dst_v.at[j - 2]],
                        sem_s.at[qn]).wait()

                @pl.when(j + 2 < GRP)
                def _():
                    pltpu.async_copy(support_hbm.at[src_v.at[j + 2]],
                                     rows.at[qn], sem_g.at[qn])
                return carry
            lax.fori_loop(0, GRP, chunk_body, 0)
            for j in (GRP - 2, GRP - 1):
                pltpu.make_async_copy(
                    rows.at[j % NBUF], acc_sh.at[dst_v.at[j]],
                    sem_s.at[j % NBUF]).wait()
            return carry0
        lax.fori_loop(0, NGRP, group_body, 0)

    with jax.named_scope("sc_writeback"):
        plsc.subcore_barrier()
        pltpu.sync_copy(acc_sh.at[pl.ds(s * ROWS_PER_SUB, ROWS_PER_SUB)],
                        out_hbm.at[c, pl.ds(s * ROWS_PER_SUB, ROWS_PER_SUB)])


_sc_segment = pl.kernel(
    _sc_body,
    out_type=jax.ShapeDtypeStruct((NC, NPAD, D), jnp.float32),
    mesh=plsc.VectorSubcoreMesh(core_axis_name="c", subcore_axis_name="s"),
    scratch_types=[
        pltpu.VMEM((GRP, CHUNK), jnp.int32),
        pltpu.VMEM((GRP, CHUNK), jnp.int32),
        pltpu.VMEM((EPW,), jnp.float32),
        pltpu.VMEM((NBUF, CHUNK, D), jnp.float32),
        pltpu.VMEM_SHARED((NPAD, D), jnp.float32),
        pltpu.SemaphoreType.DMA((NBUF,)),
        pltpu.SemaphoreType.DMA((NBUF,)),
    ],
)


BM = 1000


def _mm_first_body(x_ref, w_ref, o_ref):
    o_ref[...] = jnp.dot(x_ref[...], w_ref[...],
                         preferred_element_type=jnp.float32,
                         precision=jax.lax.Precision.HIGHEST)


def _mm_fused_body(p0_ref, p1_ref, b_ref, w_ref, o_ref):
    h = p0_ref[...] + p1_ref[...] + b_ref[...]
    h = jnp.where(h >= 0, h, 0.25 * h)
    o_ref[...] = jnp.dot(h, w_ref[...],
                         preferred_element_type=jnp.float32,
                         precision=jax.lax.Precision.HIGHEST)


def _act_body(p0_ref, p1_ref, b_ref, o_ref):
    h = p0_ref[...] + p1_ref[...] + b_ref[...]
    o_ref[...] = jnp.where(h >= 0, h, 0.25 * h)


def _mm_first(x, W):
    return pl.pallas_call(
        _mm_first_body,
        grid=(N // BM,),
        in_specs=[pl.BlockSpec((BM, D), lambda i: (i, 0)),
                  pl.BlockSpec((D, D), lambda i: (0, 0))],
        out_specs=pl.BlockSpec((BM, D), lambda i: (i, 0)),
        out_shape=jax.ShapeDtypeStruct((N, D), jnp.float32),
    )(x, W)


def _mm_fused(p0, p1, b, W):
    return pl.pallas_call(
        _mm_fused_body,
        grid=(N // BM,),
        in_specs=[pl.BlockSpec((BM, D), lambda i: (i, 0)),
                  pl.BlockSpec((BM, D), lambda i: (i, 0)),
                  pl.BlockSpec((1, D), lambda i: (0, 0)),
                  pl.BlockSpec((D, D), lambda i: (0, 0))],
        out_specs=pl.BlockSpec((BM, D), lambda i: (i, 0)),
        out_shape=jax.ShapeDtypeStruct((N, D), jnp.float32),
    )(p0, p1, b.reshape(1, D), W)


def _act(p0, p1, b):
    return pl.pallas_call(
        _act_body,
        grid=(N // BM,),
        in_specs=[pl.BlockSpec((BM, D), lambda i: (i, 0)),
                  pl.BlockSpec((BM, D), lambda i: (i, 0)),
                  pl.BlockSpec((1, D), lambda i: (0, 0))],
        out_specs=pl.BlockSpec((BM, D), lambda i: (i, 0)),
        out_shape=jax.ShapeDtypeStruct((N, D), jnp.float32),
    )(p0, p1, b.reshape(1, D))


def kernel(x, edge_index, edge_weight, W1, b1, W2, b2, W3, b3, W4, b4):
    dst = edge_index[0].astype(jnp.int32).reshape(NW, NGRP, GRP, CHUNK)
    src = edge_index[1].astype(jnp.int32).reshape(NW, NGRP, GRP, CHUNK)
    ew = edge_weight.reshape(NW, EPW)

    def seg(support):
        p = _sc_segment(support, dst, src, ew)
        return p[0, :N], p[1, :N]

    s = _mm_first(x, W1)
    p0, p1 = seg(s)
    s = _mm_fused(p0, p1, b1, W2)
    p0, p1 = seg(s)
    s = _mm_fused(p0, p1, b2, W3)
    p0, p1 = seg(s)
    s = _mm_fused(p0, p1, b3, W4)
    p0, p1 = seg(s)
    return _act(p0, p1, b4)


# speedup vs baseline: 8.4273x; 1.0353x over previous
import jax
import jax.numpy as jnp
from jax import lax
from jax.experimental import pallas as pl
from jax.experimental.pallas import tpu as pltpu
from jax.experimental.pallas import tpu_sc as plsc

N = 10000
D = 128
E = 320000
NC = 2
NS = 16
NW = NC * NS
EPW = E // NW
CHUNK = 40
NCH = EPW // CHUNK
GRP = 50
NGRP = NCH // GRP
NBUF = 4
NPAD = 10240
ROWS_PER_SUB = NPAD // NS
LANES = 16


def _sc_body(support_hbm, dst_hbm, src_hbm, ew_hbm, out_hbm,
             src_v, dst_v, ew_v, rows, acc_sh, sem_g, sem_s):
    c = lax.axis_index("c")
    s = lax.axis_index("s")
    wid = c * NS + s

    with jax.named_scope("sc_zero"):
        def zrow(r, carry):
            for k in range(D // LANES):
                rows[0, r, pl.ds(k * LANES, LANES)] = jnp.zeros((LANES,),
                                                                jnp.float32)
            return carry
        lax.fori_loop(0, CHUNK, zrow, 0)
        for t in range(ROWS_PER_SUB // CHUNK):
            pltpu.sync_copy(
                rows.at[0],
                acc_sh.at[pl.ds(s * ROWS_PER_SUB + t * CHUNK, CHUNK)])
        plsc.subcore_barrier()

    with jax.named_scope("sc_stage_ew"):
        pltpu.sync_copy(ew_hbm.at[wid], ew_v)

    def scale(rows, j):
        for g, off, lo in ((0, 0, 0), (1, 16, 0), (2, 24, 8)):
            wv = ew_v[pl.ds(j * CHUNK + off, LANES)]
            for t in range(lo, LANES):
                w = wv[t]
                e = off + t
                for k in range(D // LANES):
                    sl = pl.ds(k * LANES, LANES)
                    rows[e, sl] = rows[e, sl] * w

    with jax.named_scope("sc_edges"):
        def group_body(gi, carry0):
            pltpu.sync_copy(src_hbm.at[wid, gi], src_v)
            pltpu.sync_copy(dst_hbm.at[wid, gi], dst_v)
            pltpu.async_copy(support_hbm.at[src_v.at[0]], rows.at[0],
                             sem_g.at[0])
            pltpu.async_copy(support_hbm.at[src_v.at[1]], rows.at[1],
                             sem_g.at[1])

            def chunk_body(j, carry):
                q = j % NBUF
                qn = (j + 2) % NBUF
                pltpu.make_async_copy(
                    support_hbm.at[src_v.at[j]], rows.at[q],
                    sem_g.at[q]).wait()
                scale(rows.at[q], gi * GRP + j)
                pltpu.async_copy(rows.at[q], acc_sh.at[dst_v.at[j]],
                                 sem_s.at[q], add=True)

                @pl.when(j >= 2)
                def _():
                    pltpu.make_async_copy(
                        rows.at[qn], acc_sh.at[dst_v.at[j - 2]],
                        sem_s.at[qn]).wait()

                @pl.when(j + 2 < GRP)
                def _():
                    pltpu.async_copy(support_hbm.at[src_v.at[j + 2]],
                                     rows.at[qn], sem_g.at[qn])
                return carry
            lax.fori_loop(0, GRP, chunk_body, 0)
            for j in (GRP - 2, GRP - 1):
                pltpu.make_async_copy(
                    rows.at[j % NBUF], acc_sh.at[dst_v.at[j]],
                    sem_s.at[j % NBUF]).wait()
            return carry0
        lax.fori_loop(0, NGRP, group_body, 0)

    with jax.named_scope("sc_writeback"):
        plsc.subcore_barrier()
        pltpu.sync_copy(acc_sh.at[pl.ds(s * ROWS_PER_SUB, ROWS_PER_SUB)],
                        out_hbm.at[c, pl.ds(s * ROWS_PER_SUB, ROWS_PER_SUB)])


_sc_segment = pl.kernel(
    _sc_body,
    out_type=jax.ShapeDtypeStruct((NC, NPAD, D), jnp.float32),
    mesh=plsc.VectorSubcoreMesh(core_axis_name="c", subcore_axis_name="s"),
    scratch_types=[
        pltpu.VMEM((GRP, CHUNK), jnp.int32),
        pltpu.VMEM((GRP, CHUNK), jnp.int32),
        pltpu.VMEM((EPW,), jnp.float32),
        pltpu.VMEM((NBUF, CHUNK, D), jnp.float32),
        pltpu.VMEM_SHARED((NPAD, D), jnp.float32),
        pltpu.SemaphoreType.DMA((NBUF,)),
        pltpu.SemaphoreType.DMA((NBUF,)),
    ],
)


BM = 1000


def _mm_first_body(x_ref, w_ref, o_ref):
    o_ref[...] = jnp.dot(x_ref[...], w_ref[...],
                         preferred_element_type=jnp.float32,
                         precision=jax.lax.Precision.HIGHEST)


def _mm_fused_body(p0_ref, p1_ref, b_ref, w_ref, o_ref):
    h = p0_ref[0] + p1_ref[0] + b_ref[...]
    h = jnp.where(h >= 0, h, 0.25 * h)
    o_ref[...] = jnp.dot(h, w_ref[...],
                         preferred_element_type=jnp.float32,
                         precision=jax.lax.Precision.HIGHEST)


def _act_body(p0_ref, p1_ref, b_ref, o_ref):
    h = p0_ref[0] + p1_ref[0] + b_ref[...]
    o_ref[...] = jnp.where(h >= 0, h, 0.25 * h)


def _mm_first(x, W):
    return pl.pallas_call(
        _mm_first_body,
        grid=(N // BM,),
        in_specs=[pl.BlockSpec((BM, D), lambda i: (i, 0)),
                  pl.BlockSpec((D, D), lambda i: (0, 0))],
        out_specs=pl.BlockSpec((BM, D), lambda i: (i, 0)),
        out_shape=jax.ShapeDtypeStruct((N, D), jnp.float32),
    )(x, W)


def _mm_fused(p, b, W):
    return pl.pallas_call(
        _mm_fused_body,
        grid=(N // BM,),
        in_specs=[pl.BlockSpec((1, BM, D), lambda i: (0, i, 0)),
                  pl.BlockSpec((1, BM, D), lambda i: (1, i, 0)),
                  pl.BlockSpec((1, D), lambda i: (0, 0)),
                  pl.BlockSpec((D, D), lambda i: (0, 0))],
        out_specs=pl.BlockSpec((BM, D), lambda i: (i, 0)),
        out_shape=jax.ShapeDtypeStruct((N, D), jnp.float32),
    )(p, p, b.reshape(1, D), W)


def _act(p, b):
    return pl.pallas_call(
        _act_body,
        grid=(N // BM,),
        in_specs=[pl.BlockSpec((1, BM, D), lambda i: (0, i, 0)),
                  pl.BlockSpec((1, BM, D), lambda i: (1, i, 0)),
                  pl.BlockSpec((1, D), lambda i: (0, 0))],
        out_specs=pl.BlockSpec((BM, D), lambda i: (i, 0)),
        out_shape=jax.ShapeDtypeStruct((N, D), jnp.float32),
    )(p, p, b.reshape(1, D))


def kernel(x, edge_index, edge_weight, W1, b1, W2, b2, W3, b3, W4, b4):
    dst = edge_index[0].astype(jnp.int32).reshape(NW, NGRP, GRP, CHUNK)
    src = edge_index[1].astype(jnp.int32).reshape(NW, NGRP, GRP, CHUNK)
    ew = edge_weight.reshape(NW, EPW)

    def seg(support):
        return _sc_segment(support, dst, src, ew)

    s = _mm_first(x, W1)
    p = seg(s)
    s = _mm_fused(p, b1, W2)
    p = seg(s)
    s = _mm_fused(p, b2, W3)
    p = seg(s)
    s = _mm_fused(p, b3, W4)
    p = seg(s)
    return _act(p, b4)


# speedup vs baseline: 10.0601x; 1.1938x over previous
import jax
import jax.numpy as jnp
from jax import lax
from jax.experimental import pallas as pl
from jax.experimental.pallas import tpu as pltpu
from jax.experimental.pallas import tpu_sc as plsc

N = 10000
D = 128
E = 320000
NC = 2
NS = 16
NW = NC * NS
EPW = E // NW
CHUNK = 40
NCH = EPW // CHUNK
GRP = 50
NGRP = NCH // GRP
NBUF = 5
NPAD = 10240
ROWS_PER_SUB = NPAD // NS
LANES = 16


def _sc_body(support_hbm, dst_hbm, src_hbm, ew_hbm, out_hbm,
             src_v, dst_v, ew_v, rows, acc_sh, sem_g, sem_s):
    c = lax.axis_index("c")
    s = lax.axis_index("s")
    wid = c * NS + s

    with jax.named_scope("sc_zero"):
        def zrow(r, carry):
            for k in range(D // LANES):
                rows[0, r, pl.ds(k * LANES, LANES)] = jnp.zeros((LANES,),
                                                                jnp.float32)
            return carry
        lax.fori_loop(0, CHUNK, zrow, 0)
        for t in range(ROWS_PER_SUB // CHUNK):
            pltpu.sync_copy(
                rows.at[0],
                acc_sh.at[pl.ds(s * ROWS_PER_SUB + t * CHUNK, CHUNK)])
        plsc.subcore_barrier()

    def scale(rows, j):
        for g, off, lo in ((0, 0, 0), (1, 16, 0), (2, 24, 8)):
            wv = ew_v[j, pl.ds(off, LANES)]
            for t in range(lo, LANES):
                w = wv[t]
                e = off + t
                for k in range(D // LANES):
                    sl = pl.ds(k * LANES, LANES)
                    rows[e, sl] = rows[e, sl] * w

    with jax.named_scope("sc_edges"):
        def group_body(gi, carry0):
            pltpu.sync_copy(src_hbm.at[wid, gi], src_v)
            pltpu.sync_copy(dst_hbm.at[wid, gi], dst_v)
            pltpu.sync_copy(ew_hbm.at[wid, gi], ew_v)
            for j in range(NBUF - 2):
                pltpu.async_copy(support_hbm.at[src_v.at[j]], rows.at[j],
                                 sem_g.at[j])

            def chunk_body(j, carry):
                q = j % NBUF
                qn = (j + 3) % NBUF
                pltpu.make_async_copy(
                    support_hbm.at[src_v.at[j]], rows.at[q],
                    sem_g.at[q]).wait()
                scale(rows.at[q], j)
                pltpu.async_copy(rows.at[q], acc_sh.at[dst_v.at[j]],
                                 sem_s.at[q], add=True)

                @pl.when(j >= 2)
                def _():
                    pltpu.make_async_copy(
                        rows.at[qn], acc_sh.at[dst_v.at[j - 2]],
                        sem_s.at[qn]).wait()

                @pl.when(j + 3 < GRP)
                def _():
                    pltpu.async_copy(support_hbm.at[src_v.at[j + 3]],
                                     rows.at[qn], sem_g.at[qn])
                return carry
            lax.fori_loop(0, GRP, chunk_body, 0)
            for j in (GRP - 2, GRP - 1):
                pltpu.make_async_copy(
                    rows.at[j % NBUF], acc_sh.at[dst_v.at[j]],
                    sem_s.at[j % NBUF]).wait()
            return carry0
        lax.fori_loop(0, NGRP, group_body, 0)

    with jax.named_scope("sc_writeback"):
        plsc.subcore_barrier()
        pltpu.sync_copy(acc_sh.at[pl.ds(s * ROWS_PER_SUB, ROWS_PER_SUB)],
                        out_hbm.at[c, pl.ds(s * ROWS_PER_SUB, ROWS_PER_SUB)])


_sc_segment = pl.kernel(
    _sc_body,
    out_type=jax.ShapeDtypeStruct((NC, NPAD, D), jnp.float32),
    mesh=plsc.VectorSubcoreMesh(core_axis_name="c", subcore_axis_name="s"),
    scratch_types=[
        pltpu.VMEM((GRP, CHUNK), jnp.int32),
        pltpu.VMEM((GRP, CHUNK), jnp.int32),
        pltpu.VMEM((GRP, CHUNK), jnp.float32),
        pltpu.VMEM((NBUF, CHUNK, D), jnp.float32),
        pltpu.VMEM_SHARED((NPAD, D), jnp.float32),
        pltpu.SemaphoreType.DMA((NBUF,)),
        pltpu.SemaphoreType.DMA((NBUF,)),
    ],
)


BM = 1000


def _mm_first_body(x_ref, w_ref, o_ref):
    o_ref[...] = jnp.dot(x_ref[...], w_ref[...],
                         preferred_element_type=jnp.float32,
                         precision=jax.lax.Precision.HIGHEST)


def _mm_fused_body(p0_ref, p1_ref, b_ref, w_ref, o_ref):
    h = p0_ref[0] + p1_ref[0] + b_ref[...]
    h = jnp.where(h >= 0, h, 0.25 * h)
    o_ref[...] = jnp.dot(h, w_ref[...],
                         preferred_element_type=jnp.float32,
                         precision=jax.lax.Precision.HIGHEST)


def _act_body(p0_ref, p1_ref, b_ref, o_ref):
    h = p0_ref[0] + p1_ref[0] + b_ref[...]
    o_ref[...] = jnp.where(h >= 0, h, 0.25 * h)


def _mm_first(x, W):
    return pl.pallas_call(
        _mm_first_body,
        grid=(N // BM,),
        in_specs=[pl.BlockSpec((BM, D), lambda i: (i, 0)),
                  pl.BlockSpec((D, D), lambda i: (0, 0))],
        out_specs=pl.BlockSpec((BM, D), lambda i: (i, 0)),
        out_shape=jax.ShapeDtypeStruct((N, D), jnp.float32),
    )(x, W)


def _mm_fused(p, b, W):
    return pl.pallas_call(
        _mm_fused_body,
        grid=(N // BM,),
        in_specs=[pl.BlockSpec((1, BM, D), lambda i: (0, i, 0)),
                  pl.BlockSpec((1, BM, D), lambda i: (1, i, 0)),
                  pl.BlockSpec((1, D), lambda i: (0, 0)),
                  pl.BlockSpec((D, D), lambda i: (0, 0))],
        out_specs=pl.BlockSpec((BM, D), lambda i: (i, 0)),
        out_shape=jax.ShapeDtypeStruct((N, D), jnp.float32),
    )(p, p, b.reshape(1, D), W)


def _act(p, b):
    return pl.pallas_call(
        _act_body,
        grid=(N // BM,),
        in_specs=[pl.BlockSpec((1, BM, D), lambda i: (0, i, 0)),
                  pl.BlockSpec((1, BM, D), lambda i: (1, i, 0)),
                  pl.BlockSpec((1, D), lambda i: (0, 0))],
        out_specs=pl.BlockSpec((BM, D), lambda i: (i, 0)),
        out_shape=jax.ShapeDtypeStruct((N, D), jnp.float32),
    )(p, p, b.reshape(1, D))


def kernel(x, edge_index, edge_weight, W1, b1, W2, b2, W3, b3, W4, b4):
    dst = edge_index[0].astype(jnp.int32).reshape(NW, NGRP, GRP, CHUNK)
    src = edge_index[1].astype(jnp.int32).reshape(NW, NGRP, GRP, CHUNK)
    ew = edge_weight.reshape(NW, NGRP, GRP, CHUNK)

    def seg(support):
        return _sc_segment(support, dst, src, ew)

    s = _mm_first(x, W1)
    p = seg(s)
    s = _mm_fused(p, b1, W2)
    p = seg(s)
    s = _mm_fused(p, b2, W3)
    p = seg(s)
    s = _mm_fused(p, b3, W4)
    p = seg(s)
    return _act(p, b4)


# speedup vs baseline: 10.7568x; 1.0692x over previous
import jax
import jax.numpy as jnp
from jax import lax
from jax.experimental import pallas as pl
from jax.experimental.pallas import tpu as pltpu
from jax.experimental.pallas import tpu_sc as plsc

N = 10000
D = 128
E = 320000
NC = 2
NS = 16
NW = NC * NS
EPW = E // NW
CHUNK = 40
NCH = EPW // CHUNK
GRP = 10
NGRP = NCH // GRP
NBUF = 5
NPAD = 10240
ROWS_PER_SUB = NPAD // NS
LANES = 16


def _sc_body(support_hbm, dst_hbm, src_hbm, ew_hbm, out_hbm,
             src_v, dst_v, ew_v, rows, acc_sh, sem_g, sem_s, sem_i):
    c = lax.axis_index("c")
    s = lax.axis_index("s")
    wid = c * NS + s

    with jax.named_scope("sc_zero"):
        def zrow(r, carry):
            for k in range(D // LANES):
                rows[0, r, pl.ds(k * LANES, LANES)] = jnp.zeros((LANES,),
                                                                jnp.float32)
            return carry
        lax.fori_loop(0, CHUNK, zrow, 0)
        for t in range(ROWS_PER_SUB // CHUNK):
            pltpu.sync_copy(
                rows.at[0],
                acc_sh.at[pl.ds(s * ROWS_PER_SUB + t * CHUNK, CHUNK)])
        plsc.subcore_barrier()

    def scale(rows, gb, r):
        for g, off, lo in ((0, 0, 0), (1, 16, 0), (2, 24, 8)):
            wv = ew_v[gb, r, pl.ds(off, LANES)]
            for t in range(lo, LANES):
                w = wv[t]
                e = off + t
                for k in range(D // LANES):
                    sl = pl.ds(k * LANES, LANES)
                    rows[e, sl] = rows[e, sl] * w

    with jax.named_scope("sc_edges"):
        pltpu.sync_copy(src_hbm.at[wid, 0], src_v.at[0])
        pltpu.sync_copy(dst_hbm.at[wid, 0], dst_v.at[0])
        pltpu.sync_copy(ew_hbm.at[wid, 0], ew_v.at[0])
        for j in range(NBUF - 2):
            pltpu.async_copy(support_hbm.at[src_v.at[0, j]], rows.at[j],
                             sem_g.at[j])

        def chunk_body(j, carry):
            gi = j // GRP
            r = j % GRP
            gb = gi % 2
            q = j % NBUF
            qn = (j + 3) % NBUF
            pltpu.make_async_copy(
                support_hbm.at[src_v.at[gb, r]], rows.at[q],
                sem_g.at[q]).wait()
            scale(rows.at[q], gb, r)
            pltpu.async_copy(rows.at[q], acc_sh.at[dst_v.at[gb, r]],
                             sem_s.at[q], add=True)

            @pl.when(j >= 2)
            def _():
                jp = j - 2
                pltpu.make_async_copy(
                    rows.at[qn], acc_sh.at[dst_v.at[(jp // GRP) % 2,
                                                    jp % GRP]],
                    sem_s.at[qn]).wait()

            @pl.when((r == GRP - 3) & (gi + 1 < NGRP))
            def _():
                gbn = (gi + 1) % 2
                pltpu.make_async_copy(src_hbm.at[wid, gi + 1],
                                      src_v.at[gbn], sem_i).wait()
                pltpu.make_async_copy(dst_hbm.at[wid, gi + 1],
                                      dst_v.at[gbn], sem_i).wait()
                pltpu.make_async_copy(ew_hbm.at[wid, gi + 1],
                                      ew_v.at[gbn], sem_i).wait()

            @pl.when(j + 3 < NCH)
            def _():
                jn = j + 3
                pltpu.async_copy(
                    support_hbm.at[src_v.at[(jn // GRP) % 2, jn % GRP]],
                    rows.at[qn], sem_g.at[qn])

            @pl.when((r == 2) & (gi + 1 < NGRP))
            def _():
                gbn = (gi + 1) % 2
                pltpu.async_copy(src_hbm.at[wid, gi + 1], src_v.at[gbn],
                                 sem_i)
                pltpu.async_copy(dst_hbm.at[wid, gi + 1], dst_v.at[gbn],
                                 sem_i)
                pltpu.async_copy(ew_hbm.at[wid, gi + 1], ew_v.at[gbn],
                                 sem_i)
            return carry
        lax.fori_loop(0, NCH, chunk_body, 0)
        for j in (NCH - 2, NCH - 1):
            pltpu.make_async_copy(
                rows.at[j % NBUF],
                acc_sh.at[dst_v.at[(j // GRP) % 2, j % GRP]],
                sem_s.at[j % NBUF]).wait()

    with jax.named_scope("sc_writeback"):
        plsc.subcore_barrier()
        pltpu.sync_copy(acc_sh.at[pl.ds(s * ROWS_PER_SUB, ROWS_PER_SUB)],
                        out_hbm.at[c, pl.ds(s * ROWS_PER_SUB, ROWS_PER_SUB)])


_sc_segment = pl.kernel(
    _sc_body,
    out_type=jax.ShapeDtypeStruct((NC, NPAD, D), jnp.float32),
    mesh=plsc.VectorSubcoreMesh(core_axis_name="c", subcore_axis_name="s"),
    scratch_types=[
        pltpu.VMEM((2, GRP, CHUNK), jnp.int32),
        pltpu.VMEM((2, GRP, CHUNK), jnp.int32),
        pltpu.VMEM((2, GRP, CHUNK), jnp.float32),
        pltpu.VMEM((NBUF, CHUNK, D), jnp.float32),
        pltpu.VMEM_SHARED((NPAD, D), jnp.float32),
        pltpu.SemaphoreType.DMA((NBUF,)),
        pltpu.SemaphoreType.DMA((NBUF,)),
        pltpu.SemaphoreType.DMA,
    ],
)


BM = 1000


def _mm_first_body(x_ref, w_ref, o_ref):
    o_ref[...] = jnp.dot(x_ref[...], w_ref[...],
                         preferred_element_type=jnp.float32,
                         precision=jax.lax.Precision.HIGHEST)


def _mm_fused_body(p0_ref, p1_ref, b_ref, w_ref, o_ref):
    h = p0_ref[0] + p1_ref[0] + b_ref[...]
    h = jnp.where(h >= 0, h, 0.25 * h)
    o_ref[...] = jnp.dot(h, w_ref[...],
                         preferred_element_type=jnp.float32,
                         precision=jax.lax.Precision.HIGHEST)


def _act_body(p0_ref, p1_ref, b_ref, o_ref):
    h = p0_ref[0] + p1_ref[0] + b_ref[...]
    o_ref[...] = jnp.where(h >= 0, h, 0.25 * h)


def _mm_first(x, W):
    return pl.pallas_call(
        _mm_first_body,
        grid=(N // BM,),
        in_specs=[pl.BlockSpec((BM, D), lambda i: (i, 0)),
                  pl.BlockSpec((D, D), lambda i: (0, 0))],
        out_specs=pl.BlockSpec((BM, D), lambda i: (i, 0)),
        out_shape=jax.ShapeDtypeStruct((N, D), jnp.float32),
    )(x, W)


def _mm_fused(p, b, W):
    return pl.pallas_call(
        _mm_fused_body,
        grid=(N // BM,),
        in_specs=[pl.BlockSpec((1, BM, D), lambda i: (0, i, 0)),
                  pl.BlockSpec((1, BM, D), lambda i: (1, i, 0)),
                  pl.BlockSpec((1, D), lambda i: (0, 0)),
                  pl.BlockSpec((D, D), lambda i: (0, 0))],
        out_specs=pl.BlockSpec((BM, D), lambda i: (i, 0)),
        out_shape=jax.ShapeDtypeStruct((N, D), jnp.float32),
    )(p, p, b.reshape(1, D), W)


def _act(p, b):
    return pl.pallas_call(
        _act_body,
        grid=(N // BM,),
        in_specs=[pl.BlockSpec((1, BM, D), lambda i: (0, i, 0)),
                  pl.BlockSpec((1, BM, D), lambda i: (1, i, 0)),
                  pl.BlockSpec((1, D), lambda i: (0, 0))],
        out_specs=pl.BlockSpec((BM, D), lambda i: (i, 0)),
        out_shape=jax.ShapeDtypeStruct((N, D), jnp.float32),
    )(p, p, b.reshape(1, D))


def kernel(x, edge_index, edge_weight, W1, b1, W2, b2, W3, b3, W4, b4):
    dst = edge_index[0].astype(jnp.int32).reshape(NW, NGRP, GRP, CHUNK)
    src = edge_index[1].astype(jnp.int32).reshape(NW, NGRP, GRP, CHUNK)
    ew = edge_weight.reshape(NW, NGRP, GRP, CHUNK)

    def seg(support):
        return _sc_segment(support, dst, src, ew)

    s = _mm_first(x, W1)
    p = seg(s)
    s = _mm_fused(p, b1, W2)
    p = seg(s)
    s = _mm_fused(p, b2, W3)
    p = seg(s)
    s = _mm_fused(p, b3, W4)
    p = seg(s)
    return _act(p, b4)


# speedup vs baseline: 11.2703x; 1.0477x over previous
import jax
import jax.numpy as jnp
from jax import lax
from jax.experimental import pallas as pl
from jax.experimental.pallas import tpu as pltpu
from jax.experimental.pallas import tpu_sc as plsc

N = 10000
D = 128
E = 320000
NC = 2
NS = 16
NW = NC * NS
EPW = E // NW
CHUNK = 40
NCH = EPW // CHUNK
GRP = 10
NGRP = NCH // GRP
NBUF = 5
NPAD = 10240
ROWS_PER_SUB = NPAD // NS
LANES = 16


def _sc_body(support_hbm, dst_hbm, src_hbm, ew_hbm, out_hbm,
             src_v, dst_v, ew_v, rows, acc_sh, sem_g, sem_s, sem_i):
    c = lax.axis_index("c")
    s = lax.axis_index("s")
    wid = c * NS + s

    with jax.named_scope("sc_zero"):
        def zrow(r, carry):
            for k in range(D // LANES):
                rows[0, r, pl.ds(k * LANES, LANES)] = jnp.zeros((LANES,),
                                                                jnp.float32)
            return carry
        lax.fori_loop(0, CHUNK, zrow, 0)
        for t in range(ROWS_PER_SUB // CHUNK):
            pltpu.sync_copy(
                rows.at[0],
                acc_sh.at[pl.ds(s * ROWS_PER_SUB + t * CHUNK, CHUNK)])
        plsc.subcore_barrier()

    def scale(rows, gb, r):
        for g, off, lo in ((0, 0, 0), (1, 16, 0), (2, 24, 8)):
            wv = ew_v[gb, r, pl.ds(off, LANES)]
            for t in range(lo, LANES):
                w = wv[t]
                e = off + t
                for k in range(D // LANES):
                    sl = pl.ds(k * LANES, LANES)
                    rows[e, sl] = rows[e, sl] * w

    with jax.named_scope("sc_edges"):
        pltpu.sync_copy(src_hbm.at[wid, 0], src_v.at[0])
        pltpu.sync_copy(dst_hbm.at[wid, 0], dst_v.at[0])
        pltpu.sync_copy(ew_hbm.at[wid, 0], ew_v.at[0])
        for j in range(NBUF - 2):
            pltpu.async_copy(support_hbm.at[src_v.at[0, j]], rows.at[j],
                             sem_g.at[j])

        def chunk_body(j, carry):
            gi = j // GRP
            r = j % GRP
            gb = gi % 2
            q = j % NBUF
            qn = (j + 3) % NBUF
            pltpu.make_async_copy(
                support_hbm.at[src_v.at[gb, r]], rows.at[q],
                sem_g.at[q]).wait()

            @pl.when(j >= 2)
            def _():
                jp = j - 2
                pltpu.make_async_copy(
                    rows.at[qn], acc_sh.at[dst_v.at[(jp // GRP) % 2,
                                                    jp % GRP]],
                    sem_s.at[qn]).wait()

            @pl.when((r == GRP - 3) & (gi + 1 < NGRP))
            def _():
                gbn = (gi + 1) % 2
                pltpu.make_async_copy(src_hbm.at[wid, gi + 1],
                                      src_v.at[gbn], sem_i).wait()
                pltpu.make_async_copy(dst_hbm.at[wid, gi + 1],
                                      dst_v.at[gbn], sem_i).wait()
                pltpu.make_async_copy(ew_hbm.at[wid, gi + 1],
                                      ew_v.at[gbn], sem_i).wait()

            @pl.when(j + 3 < NCH)
            def _():
                jn = j + 3
                pltpu.async_copy(
                    support_hbm.at[src_v.at[(jn // GRP) % 2, jn % GRP]],
                    rows.at[qn], sem_g.at[qn])

            scale(rows.at[q], gb, r)
            pltpu.async_copy(rows.at[q], acc_sh.at[dst_v.at[gb, r]],
                             sem_s.at[q], add=True)

            @pl.when((r == 2) & (gi + 1 < NGRP))
            def _():
                gbn = (gi + 1) % 2
                pltpu.async_copy(src_hbm.at[wid, gi + 1], src_v.at[gbn],
                                 sem_i)
                pltpu.async_copy(dst_hbm.at[wid, gi + 1], dst_v.at[gbn],
                                 sem_i)
                pltpu.async_copy(ew_hbm.at[wid, gi + 1], ew_v.at[gbn],
                                 sem_i)
            return carry
        lax.fori_loop(0, NCH, chunk_body, 0)
        for j in (NCH - 2, NCH - 1):
            pltpu.make_async_copy(
                rows.at[j % NBUF],
                acc_sh.at[dst_v.at[(j // GRP) % 2, j % GRP]],
                sem_s.at[j % NBUF]).wait()

    with jax.named_scope("sc_writeback"):
        plsc.subcore_barrier()
        pltpu.sync_copy(acc_sh.at[pl.ds(s * ROWS_PER_SUB, ROWS_PER_SUB)],
                        out_hbm.at[c, pl.ds(s * ROWS_PER_SUB, ROWS_PER_SUB)])


_sc_segment = pl.kernel(
    _sc_body,
    out_type=jax.ShapeDtypeStruct((NC, NPAD, D), jnp.float32),
    mesh=plsc.VectorSubcoreMesh(core_axis_name="c", subcore_axis_name="s"),
    scratch_types=[
        pltpu.VMEM((2, GRP, CHUNK), jnp.int32),
        pltpu.VMEM((2, GRP, CHUNK), jnp.int32),
        pltpu.VMEM((2, GRP, CHUNK), jnp.float32),
        pltpu.VMEM((NBUF, CHUNK, D), jnp.float32),
        pltpu.VMEM_SHARED((NPAD, D), jnp.float32),
        pltpu.SemaphoreType.DMA((NBUF,)),
        pltpu.SemaphoreType.DMA((NBUF,)),
        pltpu.SemaphoreType.DMA,
    ],
)


BM = 1000


def _mm_first_body(x_ref, w_ref, o_ref):
    o_ref[...] = jnp.dot(x_ref[...], w_ref[...],
                         preferred_element_type=jnp.float32,
                         precision=jax.lax.Precision.HIGHEST)


def _mm_fused_body(p0_ref, p1_ref, b_ref, w_ref, o_ref):
    h = p0_ref[0] + p1_ref[0] + b_ref[...]
    h = jnp.where(h >= 0, h, 0.25 * h)
    o_ref[...] = jnp.dot(h, w_ref[...],
                         preferred_element_type=jnp.float32,
                         precision=jax.lax.Precision.HIGHEST)


def _act_body(p0_ref, p1_ref, b_ref, o_ref):
    h = p0_ref[0] + p1_ref[0] + b_ref[...]
    o_ref[...] = jnp.where(h >= 0, h, 0.25 * h)


def _mm_first(x, W):
    return pl.pallas_call(
        _mm_first_body,
        grid=(N // BM,),
        in_specs=[pl.BlockSpec((BM, D), lambda i: (i, 0)),
                  pl.BlockSpec((D, D), lambda i: (0, 0))],
        out_specs=pl.BlockSpec((BM, D), lambda i: (i, 0)),
        out_shape=jax.ShapeDtypeStruct((N, D), jnp.float32),
    )(x, W)


def _mm_fused(p, b, W):
    return pl.pallas_call(
        _mm_fused_body,
        grid=(N // BM,),
        in_specs=[pl.BlockSpec((1, BM, D), lambda i: (0, i, 0)),
                  pl.BlockSpec((1, BM, D), lambda i: (1, i, 0)),
                  pl.BlockSpec((1, D), lambda i: (0, 0)),
                  pl.BlockSpec((D, D), lambda i: (0, 0))],
        out_specs=pl.BlockSpec((BM, D), lambda i: (i, 0)),
        out_shape=jax.ShapeDtypeStruct((N, D), jnp.float32),
    )(p, p, b.reshape(1, D), W)


def _act(p, b):
    return pl.pallas_call(
        _act_body,
        grid=(N // BM,),
        in_specs=[pl.BlockSpec((1, BM, D), lambda i: (0, i, 0)),
                  pl.BlockSpec((1, BM, D), lambda i: (1, i, 0)),
                  pl.BlockSpec((1, D), lambda i: (0, 0))],
        out_specs=pl.BlockSpec((BM, D), lambda i: (i, 0)),
        out_shape=jax.ShapeDtypeStruct((N, D), jnp.float32),
    )(p, p, b.reshape(1, D))


def kernel(x, edge_index, edge_weight, W1, b1, W2, b2, W3, b3, W4, b4):
    dst = edge_index[0].astype(jnp.int32).reshape(NW, NGRP, GRP, CHUNK)
    src = edge_index[1].astype(jnp.int32).reshape(NW, NGRP, GRP, CHUNK)
    ew = edge_weight.reshape(NW, NGRP, GRP, CHUNK)

    def seg(support):
        return _sc_segment(support, dst, src, ew)

    s = _mm_first(x, W1)
    p = seg(s)
    s = _mm_fused(p, b1, W2)
    p = seg(s)
    s = _mm_fused(p, b2, W3)
    p = seg(s)
    s = _mm_fused(p, b3, W4)
    p = seg(s)
    return _act(p, b4)


# speedup vs baseline: 11.4623x; 1.0170x over previous
import jax
import jax.numpy as jnp
from jax import lax
from jax.experimental import pallas as pl
from jax.experimental.pallas import tpu as pltpu
from jax.experimental.pallas import tpu_sc as plsc

N = 10000
D = 128
E = 320000
NC = 2
NS = 16
NW = NC * NS
EPW = E // NW
CHUNK = 40
NCH = EPW // CHUNK
GRP = 10
NGRP = NCH // GRP
NBUF = 5
NPAD = 10240
ROWS_PER_SUB = NPAD // NS
LANES = 16


def _sc_body(support_hbm, dst_hbm, src_hbm, ew_hbm, out_hbm,
             src_v, dst_v, ew_v, rows, acc_sh, sem_g, sem_s, sem_i):
    c = lax.axis_index("c")
    s = lax.axis_index("s")
    wid = c * NS + s

    with jax.named_scope("sc_zero"):
        def zrow(r, carry):
            for k in range(D // LANES):
                rows[0, r, pl.ds(k * LANES, LANES)] = jnp.zeros((LANES,),
                                                                jnp.float32)
            return carry
        lax.fori_loop(0, CHUNK, zrow, 0)
        for t in range(ROWS_PER_SUB // CHUNK):
            pltpu.async_copy(
                rows.at[0],
                acc_sh.at[pl.ds(s * ROWS_PER_SUB + t * CHUNK, CHUNK)], sem_i)

    def scale(rows, gb, r):
        for g, off, lo in ((0, 0, 0), (1, 16, 0), (2, 24, 8)):
            wv = ew_v[gb, r, pl.ds(off, LANES)]
            for t in range(lo, LANES):
                w = wv[t]
                e = off + t
                for k in range(D // LANES):
                    sl = pl.ds(k * LANES, LANES)
                    rows[e, sl] = rows[e, sl] * w

    with jax.named_scope("sc_edges"):
        pltpu.sync_copy(src_hbm.at[wid, 0], src_v.at[0])
        pltpu.sync_copy(dst_hbm.at[wid, 0], dst_v.at[0])
        pltpu.sync_copy(ew_hbm.at[wid, 0], ew_v.at[0])
        for t in range(ROWS_PER_SUB // CHUNK):
            pltpu.make_async_copy(
                rows.at[0],
                acc_sh.at[pl.ds(s * ROWS_PER_SUB + t * CHUNK, CHUNK)],
                sem_i).wait()
        plsc.subcore_barrier()
        for j in range(NBUF - 2):
            pltpu.async_copy(support_hbm.at[src_v.at[0, j]], rows.at[j],
                             sem_g.at[j])

        def chunk_body(j, carry):
            gi = j // GRP
            r = j % GRP
            gb = gi % 2
            q = j % NBUF
            qn = (j + 3) % NBUF
            pltpu.make_async_copy(
                support_hbm.at[src_v.at[gb, r]], rows.at[q],
                sem_g.at[q]).wait()

            @pl.when(j >= 2)
            def _():
                jp = j - 2
                pltpu.make_async_copy(
                    rows.at[qn], acc_sh.at[dst_v.at[(jp // GRP) % 2,
                                                    jp % GRP]],
                    sem_s.at[qn]).wait()

            @pl.when((r == GRP - 3) & (gi + 1 < NGRP))
            def _():
                gbn = (gi + 1) % 2
                pltpu.make_async_copy(src_hbm.at[wid, gi + 1],
                                      src_v.at[gbn], sem_i).wait()
                pltpu.make_async_copy(dst_hbm.at[wid, gi + 1],
                                      dst_v.at[gbn], sem_i).wait()
                pltpu.make_async_copy(ew_hbm.at[wid, gi + 1],
                                      ew_v.at[gbn], sem_i).wait()

            @pl.when(j + 3 < NCH)
            def _():
                jn = j + 3
                pltpu.async_copy(
                    support_hbm.at[src_v.at[(jn // GRP) % 2, jn % GRP]],
                    rows.at[qn], sem_g.at[qn])

            scale(rows.at[q], gb, r)
            pltpu.async_copy(rows.at[q], acc_sh.at[dst_v.at[gb, r]],
                             sem_s.at[q], add=True)

            @pl.when((r == 2) & (gi + 1 < NGRP))
            def _():
                gbn = (gi + 1) % 2
                pltpu.async_copy(src_hbm.at[wid, gi + 1], src_v.at[gbn],
                                 sem_i)
                pltpu.async_copy(dst_hbm.at[wid, gi + 1], dst_v.at[gbn],
                                 sem_i)
                pltpu.async_copy(ew_hbm.at[wid, gi + 1], ew_v.at[gbn],
                                 sem_i)
            return carry
        lax.fori_loop(0, NCH, chunk_body, 0)
        for j in (NCH - 2, NCH - 1):
            pltpu.make_async_copy(
                rows.at[j % NBUF],
                acc_sh.at[dst_v.at[(j // GRP) % 2, j % GRP]],
                sem_s.at[j % NBUF]).wait()

    with jax.named_scope("sc_writeback"):
        plsc.subcore_barrier()
        pltpu.sync_copy(acc_sh.at[pl.ds(s * ROWS_PER_SUB, ROWS_PER_SUB)],
                        out_hbm.at[c, pl.ds(s * ROWS_PER_SUB, ROWS_PER_SUB)])


_sc_segment = pl.kernel(
    _sc_body,
    out_type=jax.ShapeDtypeStruct((NC, NPAD, D), jnp.float32),
    mesh=plsc.VectorSubcoreMesh(core_axis_name="c", subcore_axis_name="s"),
    scratch_types=[
        pltpu.VMEM((2, GRP, CHUNK), jnp.int32),
        pltpu.VMEM((2, GRP, CHUNK), jnp.int32),
        pltpu.VMEM((2, GRP, CHUNK), jnp.float32),
        pltpu.VMEM((NBUF, CHUNK, D), jnp.float32),
        pltpu.VMEM_SHARED((NPAD, D), jnp.float32),
        pltpu.SemaphoreType.DMA((NBUF,)),
        pltpu.SemaphoreType.DMA((NBUF,)),
        pltpu.SemaphoreType.DMA,
    ],
)


BM = 1000


def _mm_first_body(x_ref, w_ref, o_ref):
    o_ref[...] = jnp.dot(x_ref[...], w_ref[...],
                         preferred_element_type=jnp.float32,
                         precision=jax.lax.Precision.HIGHEST)


def _mm_fused_body(p0_ref, p1_ref, b_ref, w_ref, o_ref):
    h = p0_ref[0] + p1_ref[0] + b_ref[...]
    h = jnp.where(h >= 0, h, 0.25 * h)
    o_ref[...] = jnp.dot(h, w_ref[...],
                         preferred_element_type=jnp.float32,
                         precision=jax.lax.Precision.HIGHEST)


def _act_body(p0_ref, p1_ref, b_ref, o_ref):
    h = p0_ref[0] + p1_ref[0] + b_ref[...]
    o_ref[...] = jnp.where(h >= 0, h, 0.25 * h)


def _mm_first(x, W):
    return pl.pallas_call(
        _mm_first_body,
        grid=(N // BM,),
        in_specs=[pl.BlockSpec((BM, D), lambda i: (i, 0)),
                  pl.BlockSpec((D, D), lambda i: (0, 0))],
        out_specs=pl.BlockSpec((BM, D), lambda i: (i, 0)),
        out_shape=jax.ShapeDtypeStruct((N, D), jnp.float32),
    )(x, W)


def _mm_fused(p, b, W):
    return pl.pallas_call(
        _mm_fused_body,
        grid=(N // BM,),
        in_specs=[pl.BlockSpec((1, BM, D), lambda i: (0, i, 0)),
                  pl.BlockSpec((1, BM, D), lambda i: (1, i, 0)),
                  pl.BlockSpec((1, D), lambda i: (0, 0)),
                  pl.BlockSpec((D, D), lambda i: (0, 0))],
        out_specs=pl.BlockSpec((BM, D), lambda i: (i, 0)),
        out_shape=jax.ShapeDtypeStruct((N, D), jnp.float32),
    )(p, p, b.reshape(1, D), W)


def _act(p, b):
    return pl.pallas_call(
        _act_body,
        grid=(N // BM,),
        in_specs=[pl.BlockSpec((1, BM, D), lambda i: (0, i, 0)),
                  pl.BlockSpec((1, BM, D), lambda i: (1, i, 0)),
                  pl.BlockSpec((1, D), lambda i: (0, 0))],
        out_specs=pl.BlockSpec((BM, D), lambda i: (i, 0)),
        out_shape=jax.ShapeDtypeStruct((N, D), jnp.float32),
    )(p, p, b.reshape(1, D))


def kernel(x, edge_index, edge_weight, W1, b1, W2, b2, W3, b3, W4, b4):
    dst = edge_index[0].astype(jnp.int32).reshape(NW, NGRP, GRP, CHUNK)
    src = edge_index[1].astype(jnp.int32).reshape(NW, NGRP, GRP, CHUNK)
    ew = edge_weight.reshape(NW, NGRP, GRP, CHUNK)

    def seg(support):
        return _sc_segment(support, dst, src, ew)

    s = _mm_first(x, W1)
    p = seg(s)
    s = _mm_fused(p, b1, W2)
    p = seg(s)
    s = _mm_fused(p, b2, W3)
    p = seg(s)
    s = _mm_fused(p, b3, W4)
    p = seg(s)
    return _act(p, b4)
